# baseline (speedup 1.0000x reference)
import functools

import jax
import jax.numpy as jnp
from jax import lax
from jax.experimental import pallas as pl
from jax.experimental.pallas import tpu as pltpu
from jax.experimental.pallas import tpu_sc as plsc

D_MODEL = 64
NUM_WORKERS = 32
CHUNK = 128


def _build_kernel(batch: int):
    assert batch % NUM_WORKERS == 0
    b_per_w = batch // NUM_WORKERS
    assert b_per_w % CHUNK == 0
    n_chunks = b_per_w // CHUNK

    mesh = plsc.VectorSubcoreMesh(core_axis_name="c", subcore_axis_name="s")

    @functools.partial(
        pl.kernel,
        out_type=jax.ShapeDtypeStruct((batch, D_MODEL), jnp.float32),
        mesh=mesh,
        scratch_types=[
            pltpu.VMEM((b_per_w,), jnp.int32),
            pltpu.VMEM((CHUNK, D_MODEL), jnp.float32),
            pltpu.SemaphoreType.DMA,
        ],
        compiler_params=pltpu.CompilerParams(use_tc_tiling_on_sc=False),
    )
    def emb_kernel(table_hbm, idx_hbm, out_hbm, idx_v, rows_v, sem):
        wid = lax.axis_index("s") * 2 + lax.axis_index("c")
        base = wid * b_per_w
        pltpu.sync_copy(idx_hbm.at[pl.ds(base, b_per_w)], idx_v)

        @pl.loop(0, n_chunks)
        def _(j):
            off = j * CHUNK
            pltpu.async_copy(
                table_hbm.at[idx_v.at[pl.ds(off, CHUNK)]], rows_v, sem
            ).wait()
            pltpu.sync_copy(rows_v, out_hbm.at[pl.ds(base + off, CHUNK)])

    return emb_kernel


def kernel(x, emb_table):
    b, s = x.shape
    flat_idx = x.reshape(b * s).astype(jnp.int32)
    out = _build_kernel(b * s)(emb_table, flat_idx)
    return out.reshape(b, s, D_MODEL)


# speedup vs baseline: 1.0868x; 1.0868x over previous
import functools

import jax
import jax.numpy as jnp
from jax import lax
from jax.experimental import pallas as pl
from jax.experimental.pallas import tpu as pltpu
from jax.experimental.pallas import tpu_sc as plsc

D_MODEL = 64
NUM_WORKERS = 32
CHUNK = 512


def _build_kernel(batch: int):
    assert batch % NUM_WORKERS == 0
    b_per_w = batch // NUM_WORKERS
    assert b_per_w % CHUNK == 0
    n_chunks = b_per_w // CHUNK

    mesh = plsc.VectorSubcoreMesh(core_axis_name="c", subcore_axis_name="s")

    @functools.partial(
        pl.kernel,
        out_type=jax.ShapeDtypeStruct((batch, D_MODEL), jnp.float32),
        mesh=mesh,
        scratch_types=[
            pltpu.VMEM((b_per_w,), jnp.int32),
            pltpu.VMEM((CHUNK, D_MODEL), jnp.float32),
            pltpu.SemaphoreType.DMA,
        ],
        compiler_params=pltpu.CompilerParams(use_tc_tiling_on_sc=False),
    )
    def emb_kernel(table_hbm, idx_hbm, out_hbm, idx_v, rows_v, sem):
        wid = lax.axis_index("s") * 2 + lax.axis_index("c")
        base = wid * b_per_w
        pltpu.sync_copy(idx_hbm.at[pl.ds(base, b_per_w)], idx_v)

        @pl.loop(0, n_chunks)
        def _(j):
            off = j * CHUNK
            pltpu.async_copy(
                table_hbm.at[idx_v.at[pl.ds(off, CHUNK)]], rows_v, sem
            ).wait()
            pltpu.sync_copy(rows_v, out_hbm.at[pl.ds(base + off, CHUNK)])

    return emb_kernel


def kernel(x, emb_table):
    b, s = x.shape
    flat_idx = x.reshape(b * s).astype(jnp.int32)
    out = _build_kernel(b * s)(emb_table, flat_idx)
    return out.reshape(b, s, D_MODEL)


# speedup vs baseline: 1.1141x; 1.0251x over previous
import functools

import jax
import jax.numpy as jnp
from jax import lax
from jax.experimental import pallas as pl
from jax.experimental.pallas import tpu as pltpu
from jax.experimental.pallas import tpu_sc as plsc

D_MODEL = 64
NUM_WORKERS = 32
SUPER = 400
NBUF = 4


def _build_kernel(batch: int):
    assert batch % NUM_WORKERS == 0
    b_per_w = batch // NUM_WORKERS
    assert b_per_w % SUPER == 0
    n_super = b_per_w // SUPER
    assert n_super % NBUF == 0

    mesh = plsc.VectorSubcoreMesh(core_axis_name="c", subcore_axis_name="s")

    @functools.partial(
        pl.kernel,
        out_type=jax.ShapeDtypeStruct((batch, D_MODEL), jnp.float32),
        mesh=mesh,
        scratch_types=[
            pltpu.VMEM((b_per_w,), jnp.int32),
            [pltpu.VMEM((SUPER, D_MODEL), jnp.float32) for _ in range(NBUF)],
            [pltpu.SemaphoreType.DMA for _ in range(NBUF)],
            [pltpu.SemaphoreType.DMA for _ in range(NBUF)],
        ],
        compiler_params=pltpu.CompilerParams(use_tc_tiling_on_sc=False),
    )
    def emb_kernel(table_hbm, idx_hbm, out_hbm, idx_v, rows, gsem, wsem):
        wid = lax.axis_index("s") * 2 + lax.axis_index("c")
        base = wid * b_per_w
        pltpu.sync_copy(idx_hbm.at[pl.ds(base, b_per_w)], idx_v)

        def fire_gather(t, b):
            pltpu.async_copy(
                table_hbm.at[idx_v.at[pl.ds(t * SUPER, SUPER)]], rows[b], gsem[b]
            )

        def drain_gather(t, b):
            pltpu.make_async_copy(
                table_hbm.at[idx_v.at[pl.ds(t * SUPER, SUPER)]], rows[b], gsem[b]
            ).wait()

        def fire_write(t, b):
            pltpu.async_copy(
                rows[b], out_hbm.at[pl.ds(base + t * SUPER, SUPER)], wsem[b]
            )

        def drain_write(t, b):
            pltpu.make_async_copy(
                rows[b], out_hbm.at[pl.ds(base + t * SUPER, SUPER)], wsem[b]
            ).wait()

        fire_gather(0, 0)
        fire_gather(1, 1)

        @pl.loop(0, n_super, step=NBUF)
        def _(t_base):
            for b in range(NBUF):
                t = t_base + b
                drain_gather(t, b)
                fire_write(t, b)
                b2 = (b + 2) % NBUF

                @pl.when(t >= 2)
                def _():
                    drain_write(t - 2, b2)

                @pl.when(t + 2 < n_super)
                def _():
                    fire_gather(t + 2, b2)

        drain_write(n_super - 2, (n_super - 2) % NBUF)
        drain_write(n_super - 1, (n_super - 1) % NBUF)

    return emb_kernel


def kernel(x, emb_table):
    b, s = x.shape
    flat_idx = x.reshape(b * s).astype(jnp.int32)
    out = _build_kernel(b * s)(emb_table, flat_idx)
    return out.reshape(b, s, D_MODEL)
